# Initial kernel scaffold; baseline (speedup 1.0000x reference)
import jax
import jax.numpy as jnp
from jax.experimental import pallas as pl


def kernel(node_scalars, edge_index, edge_feats, W1, b1, W2, b2):
    raise NotImplementedError("write your pallas kernel here")


# speedup vs baseline: 3.1271x; 3.1271x over previous
import functools

import jax
import jax.numpy as jnp
from jax import lax
from jax.experimental import pallas as pl
from jax.experimental.pallas import tpu as pltpu
from jax.experimental.pallas import tpu_sc as plsc

N_NODES = 10000
N_EDGES = 320000
D_SCALAR = 128
D_EDGE = 16
D_HIDDEN = 128

_A_BLK = 1000


def _tables_body(ns_ref, w_ref, out_ref):
    out_ref[...] = jnp.dot(ns_ref[...], w_ref[0],
                           preferred_element_type=jnp.float32)


def _node_tables(node_scalars, w1_nodes_stacked):
    return pl.pallas_call(
        _tables_body,
        grid=(2, N_NODES // _A_BLK),
        in_specs=[
            pl.BlockSpec((_A_BLK, D_SCALAR), lambda t, i: (i, 0)),
            pl.BlockSpec((1, D_SCALAR, D_HIDDEN), lambda t, i: (t, 0, 0)),
        ],
        out_specs=pl.BlockSpec((_A_BLK, D_HIDDEN),
                               lambda t, i: (t * (N_NODES // _A_BLK) + i, 0)),
        out_shape=jax.ShapeDtypeStruct((2 * N_NODES, D_HIDDEN), jnp.float32),
    )(node_scalars, w1_nodes_stacked)


_NW = 32
_EP = N_EDGES // _NW
_C = 400
_NCHUNK = _EP // _C


_GATHER_ADD_CACHE = []


def _gather_add_build():
    if _GATHER_ADD_CACHE:
        return _GATHER_ADD_CACHE[0]
    mesh = plsc.VectorSubcoreMesh(core_axis_name="c", subcore_axis_name="s")

    @functools.partial(
        pl.kernel,
        out_type=jax.ShapeDtypeStruct((N_EDGES, D_HIDDEN), jnp.float32),
        mesh=mesh,
        scratch_types=[
            pltpu.VMEM((_C,), jnp.int32),
            pltpu.VMEM((_C,), jnp.int32),
            pltpu.VMEM((_C, D_HIDDEN), jnp.float32),
            pltpu.VMEM((_C, D_HIDDEN), jnp.float32),
            pltpu.SemaphoreType.DMA,
            pltpu.SemaphoreType.DMA,
        ],
    )
    def gather_add(table_hbm, src_hbm, dst_hbm, out_hbm,
                   idx_s, idx_d, rows_a, rows_b, sem_a, sem_b):
        wid = lax.axis_index("s") * 2 + lax.axis_index("c")
        base = wid * _EP

        def chunk(ci, carry):
            off = pl.multiple_of(base + ci * _C, 8)
            pltpu.sync_copy(src_hbm.at[pl.ds(off, _C)], idx_s)
            pltpu.sync_copy(dst_hbm.at[pl.ds(off, _C)], idx_d)
            cp_a = pltpu.async_copy(table_hbm.at[idx_s], rows_a, sem_a)
            cp_b = pltpu.async_copy(table_hbm.at[idx_d], rows_b, sem_b)
            cp_a.wait()
            cp_b.wait()

            def add_row(i, c2):
                for j in range(D_HIDDEN // 16):
                    sl = pl.ds(j * 16, 16)
                    rows_a[i, sl] = rows_a[i, sl] + rows_b[i, sl]
                return c2

            lax.fori_loop(0, _C, add_row, 0)
            pltpu.sync_copy(rows_a, out_hbm.at[pl.ds(off, _C)])
            return carry

        lax.fori_loop(0, _NCHUNK, chunk, 0)

    _GATHER_ADD_CACHE.append(gather_add)
    return gather_add


_E_BLK = 4000


def _tail_body(g_ref, ef_ref, w1e_ref, b1_ref, w2_ref, b2_ref, out_ref):
    x = (g_ref[...]
         + jnp.dot(ef_ref[...], w1e_ref[...],
                   preferred_element_type=jnp.float32)
         + b1_ref[...])
    h = x * jax.nn.sigmoid(x)
    out_ref[...] = (jnp.dot(h, w2_ref[...],
                            preferred_element_type=jnp.float32)
                    + b2_ref[...])


def _mlp_tail(g, edge_feats, w1e, b1, w2, b2):
    nblk = N_EDGES // _E_BLK
    return pl.pallas_call(
        _tail_body,
        grid=(nblk,),
        in_specs=[
            pl.BlockSpec((_E_BLK, D_HIDDEN), lambda i: (i, 0)),
            pl.BlockSpec((_E_BLK, D_EDGE), lambda i: (i, 0)),
            pl.BlockSpec((D_EDGE, D_HIDDEN), lambda i: (0, 0)),
            pl.BlockSpec((1, D_HIDDEN), lambda i: (0, 0)),
            pl.BlockSpec((D_HIDDEN, D_EDGE), lambda i: (0, 0)),
            pl.BlockSpec((1, D_EDGE), lambda i: (0, 0)),
        ],
        out_specs=pl.BlockSpec((_E_BLK, D_EDGE), lambda i: (i, 0)),
        out_shape=jax.ShapeDtypeStruct((N_EDGES, D_EDGE), jnp.float32),
    )(g, edge_feats, w1e, b1, w2, b2)


def kernel(node_scalars, edge_index, edge_feats, W1, b1, W2, b2):
    src = edge_index[0].astype(jnp.int32)
    dst = edge_index[1].astype(jnp.int32) + jnp.int32(N_NODES)
    w1_nodes = jnp.stack([W1[:D_SCALAR], W1[D_SCALAR:2 * D_SCALAR]])
    w1e = W1[2 * D_SCALAR:]
    table = _node_tables(node_scalars, w1_nodes)
    g = _gather_add_build()(table, src, dst)
    return _mlp_tail(g, edge_feats, w1e,
                     b1.reshape(1, D_HIDDEN), W2, b2.reshape(1, D_EDGE))


# speedup vs baseline: 3.6447x; 1.1655x over previous
import functools

import jax
import jax.numpy as jnp
from jax import lax
from jax.experimental import pallas as pl
from jax.experimental.pallas import tpu as pltpu
from jax.experimental.pallas import tpu_sc as plsc

N_NODES = 10000
N_EDGES = 320000
D_SCALAR = 128
D_EDGE = 16
D_HIDDEN = 128

_A_BLK = 1000


def _tables_body(ns_ref, w_ref, out_ref):
    out_ref[...] = jnp.dot(ns_ref[...], w_ref[0],
                           preferred_element_type=jnp.float32)


def _node_tables(node_scalars, w1_nodes_stacked):
    return pl.pallas_call(
        _tables_body,
        grid=(2, N_NODES // _A_BLK),
        in_specs=[
            pl.BlockSpec((_A_BLK, D_SCALAR), lambda t, i: (i, 0)),
            pl.BlockSpec((1, D_SCALAR, D_HIDDEN), lambda t, i: (t, 0, 0)),
        ],
        out_specs=pl.BlockSpec((_A_BLK, D_HIDDEN),
                               lambda t, i: (t * (N_NODES // _A_BLK) + i, 0)),
        out_shape=jax.ShapeDtypeStruct((2 * N_NODES, D_HIDDEN), jnp.float32),
    )(node_scalars, w1_nodes_stacked)


_NW = 32
_EP = N_EDGES // _NW
_C = 400
_NCHUNK = _EP // _C


_GATHER_ADD_CACHE = []


def _gather_add_build():
    if _GATHER_ADD_CACHE:
        return _GATHER_ADD_CACHE[0]
    mesh = plsc.VectorSubcoreMesh(core_axis_name="c", subcore_axis_name="s")

    @functools.partial(
        pl.kernel,
        out_type=jax.ShapeDtypeStruct((N_EDGES, D_HIDDEN), jnp.float32),
        mesh=mesh,
        scratch_types=[
            pltpu.VMEM((_EP,), jnp.int32),
            pltpu.VMEM((_EP,), jnp.int32),
            pltpu.VMEM((_C, D_HIDDEN), jnp.float32),
            pltpu.VMEM((_C, D_HIDDEN), jnp.float32),
            pltpu.SemaphoreType.DMA,
            pltpu.SemaphoreType.DMA,
            pltpu.SemaphoreType.DMA,
            pltpu.SemaphoreType.DMA,
        ],
    )
    def gather_add(table_hbm, src_hbm, dst_hbm, out_hbm,
                   idx_s, idx_d, buf0, buf1, gs0, gs1, ws0, ws1):
        wid = lax.axis_index("s") * 2 + lax.axis_index("c")
        base = pl.multiple_of(wid * _EP, 8)
        bufs = (buf0, buf1)
        gsems = (gs0, gs1)
        wsems = (ws0, ws1)

        pltpu.sync_copy(src_hbm.at[pl.ds(base, _EP)], idx_s)
        pltpu.sync_copy(dst_hbm.at[pl.ds(base, _EP)], idx_d)

        def g1(ci):
            s = ci % 2
            return pltpu.async_copy(
                table_hbm.at[idx_s.at[pl.ds(ci * _C, _C)]], bufs[s], gsems[s])

        def g2(ci):
            s = ci % 2
            return pltpu.async_copy(
                table_hbm.at[idx_d.at[pl.ds(ci * _C, _C)]], bufs[s], gsems[s],
                add=True)

        def wb(ci):
            s = ci % 2
            return pltpu.async_copy(
                bufs[s], out_hbm.at[pl.ds(base + ci * _C, _C)], wsems[s])

        wbd = [None] * _NCHUNK
        d = g1(0)
        for ci in range(_NCHUNK):
            d.wait()
            dg2 = g2(ci)
            if ci >= 1:
                wbd[ci - 1].wait()
            if ci + 1 < _NCHUNK:
                d = g1(ci + 1)
            dg2.wait()
            wbd[ci] = wb(ci)
        wbd[_NCHUNK - 1].wait()

    _GATHER_ADD_CACHE.append(gather_add)
    return gather_add


_E_BLK = 8000


def _tail_body(g_ref, ef_ref, w1e_ref, b1_ref, w2_ref, b2_ref, out_ref):
    x = (g_ref[...]
         + jnp.dot(ef_ref[...], w1e_ref[...],
                   preferred_element_type=jnp.float32)
         + b1_ref[...])
    h = x * jax.nn.sigmoid(x)
    out_ref[...] = (jnp.dot(h, w2_ref[...],
                            preferred_element_type=jnp.float32)
                    + b2_ref[...])


def _mlp_tail(g, edge_feats, w1e, b1, w2, b2):
    nblk = N_EDGES // _E_BLK
    return pl.pallas_call(
        _tail_body,
        grid=(nblk,),
        in_specs=[
            pl.BlockSpec((_E_BLK, D_HIDDEN), lambda i: (i, 0)),
            pl.BlockSpec((_E_BLK, D_EDGE), lambda i: (i, 0)),
            pl.BlockSpec((D_EDGE, D_HIDDEN), lambda i: (0, 0)),
            pl.BlockSpec((1, D_HIDDEN), lambda i: (0, 0)),
            pl.BlockSpec((D_HIDDEN, D_EDGE), lambda i: (0, 0)),
            pl.BlockSpec((1, D_EDGE), lambda i: (0, 0)),
        ],
        out_specs=pl.BlockSpec((_E_BLK, D_EDGE), lambda i: (i, 0)),
        out_shape=jax.ShapeDtypeStruct((N_EDGES, D_EDGE), jnp.float32),
    )(g, edge_feats, w1e, b1, w2, b2)


def kernel(node_scalars, edge_index, edge_feats, W1, b1, W2, b2):
    src = edge_index[0].astype(jnp.int32)
    dst = edge_index[1].astype(jnp.int32) + jnp.int32(N_NODES)
    w1_nodes = jnp.stack([W1[:D_SCALAR], W1[D_SCALAR:2 * D_SCALAR]])
    w1e = W1[2 * D_SCALAR:]
    table = _node_tables(node_scalars, w1_nodes)
    g = _gather_add_build()(table, src, dst)
    return _mlp_tail(g, edge_feats, w1e,
                     b1.reshape(1, D_HIDDEN), W2, b2.reshape(1, D_EDGE))
